# Initial kernel scaffold; baseline (speedup 1.0000x reference)
import jax
import jax.numpy as jnp
from jax.experimental import pallas as pl


def kernel(x, time_delta, W, b):
    raise NotImplementedError("write your pallas kernel here")


# speedup vs baseline: 12.6757x; 12.6757x over previous
import functools

import jax
import jax.numpy as jnp
from jax.experimental import pallas as pl
from jax.experimental.pallas import tpu as pltpu

INPUT_DIM = 512
D_MODEL = 768
PATCH_SIZE = 7.0
MAX_PATCHES = 512
T = 4096
TCHUNK = 512
NCHUNK = T // TCHUNK


def _patch_kernel(ids_row_ref, ids_chunk_ref, x_ref, w_ref, b_ref,
                  out_ref, mask_ref, acc_ref, cnt_ref):
    c = pl.program_id(1)

    lo = ids_row_ref[0, 0, 0]
    hi = ids_row_ref[0, 0, T - 1]
    num = hi - lo + 1
    base_id = hi - (MAX_PATCHES - 1)

    @pl.when(c == 0)
    def _init():
        acc_ref[...] = jnp.zeros_like(acc_ref)
        cnt_ref[...] = jnp.zeros_like(cnt_ref)

    ids_chunk = ids_chunk_ref[0, 0]
    r_iota = jax.lax.broadcasted_iota(jnp.int32, (MAX_PATCHES, TCHUNK), 0)
    sel = (ids_chunk == base_id + r_iota)
    s = sel.astype(jnp.float32)

    xc = x_ref[0]
    acc_ref[...] += jnp.dot(s, xc, preferred_element_type=jnp.float32)
    cnt_ref[...] += jnp.sum(s, axis=1, keepdims=True)

    @pl.when(c == NCHUNK - 1)
    def _finish():
        feats = acc_ref[...] / jnp.maximum(cnt_ref[...], 1.0)
        proj = jax.lax.dot_general(
            feats, w_ref[...],
            dimension_numbers=(((1,), (1,)), ((), ())),
            preferred_element_type=jnp.float32,
        )
        out_ref[0] = proj + b_ref[...]
        lane = jax.lax.broadcasted_iota(jnp.int32, (1, MAX_PATCHES), 1)
        mask_ref[0] = ((num - MAX_PATCHES + lane) >= 0).astype(jnp.int32)


@jax.jit
def kernel(x, time_delta, W, b):
    B = x.shape[0]
    t = jnp.cumsum(time_delta, axis=1)
    ids = jnp.floor(t / PATCH_SIZE).astype(jnp.int32)

    ids_row = ids.reshape(B, 1, T)
    ids_chunk = ids.reshape(B, NCHUNK, 1, TCHUNK)
    b2 = b.reshape(1, D_MODEL)

    out, mask_i32 = pl.pallas_call(
        _patch_kernel,
        grid=(B, NCHUNK),
        in_specs=[
            pl.BlockSpec((1, 1, T), lambda i, c: (i, 0, 0)),
            pl.BlockSpec((1, 1, 1, TCHUNK), lambda i, c: (i, c, 0, 0)),
            pl.BlockSpec((1, TCHUNK, INPUT_DIM), lambda i, c: (i, c, 0)),
            pl.BlockSpec((D_MODEL, INPUT_DIM), lambda i, c: (0, 0)),
            pl.BlockSpec((1, D_MODEL), lambda i, c: (0, 0)),
        ],
        out_specs=[
            pl.BlockSpec((1, MAX_PATCHES, D_MODEL), lambda i, c: (i, 0, 0)),
            pl.BlockSpec((1, 1, MAX_PATCHES), lambda i, c: (i, 0, 0)),
        ],
        out_shape=[
            jax.ShapeDtypeStruct((B, MAX_PATCHES, D_MODEL), jnp.float32),
            jax.ShapeDtypeStruct((B, 1, MAX_PATCHES), jnp.int32),
        ],
        scratch_shapes=[
            pltpu.VMEM((MAX_PATCHES, INPUT_DIM), jnp.float32),
            pltpu.VMEM((MAX_PATCHES, 1), jnp.float32),
        ],
    )(ids_row, ids_chunk, x, W, b2)

    masks = mask_i32.reshape(B, MAX_PATCHES) != 0
    return out, masks


# speedup vs baseline: 13.9154x; 1.0978x over previous
import functools

import jax
import jax.numpy as jnp
from jax.experimental import pallas as pl
from jax.experimental.pallas import tpu as pltpu

INPUT_DIM = 512
D_MODEL = 768
PATCH_SIZE = 7.0
MAX_PATCHES = 512
T = 4096
TCHUNK = 512
NCHUNK = T // TCHUNK
BAND = 96


def _patch_kernel(ids_row_ref, ids_chunk_ref, x_ref, w_ref, b_ref,
                  out_ref, mask_ref, acc_ref, cnt_ref):
    c = pl.program_id(1)

    lo = ids_row_ref[0, 0, 0]
    hi = ids_row_ref[0, 0, T - 1]
    num = hi - lo + 1
    base_id = hi - (MAX_PATCHES - 1)

    @pl.when(c == 0)
    def _init():
        acc_ref[...] = jnp.zeros_like(acc_ref)
        cnt_ref[...] = jnp.zeros_like(cnt_ref)

    ids_chunk = ids_chunk_ref[0, 0]
    first_rel = ids_chunk_ref[0, 0, 0, 0] - base_id
    start = jnp.clip((first_rel // 8) * 8, 0, MAX_PATCHES - BAND)
    start = pl.multiple_of(start, 8)
    r_iota = jax.lax.broadcasted_iota(jnp.int32, (BAND, TCHUNK), 0)
    sel = (ids_chunk == base_id + start + r_iota)
    s = sel.astype(jnp.float32)

    xc = x_ref[0]
    acc_ref[pl.ds(start, BAND), :] += jnp.dot(
        s, xc, preferred_element_type=jnp.float32)
    cnt_ref[pl.ds(start, BAND), :] += jnp.sum(s, axis=1, keepdims=True)

    @pl.when(c == NCHUNK - 1)
    def _finish():
        feats = acc_ref[...] / jnp.maximum(cnt_ref[...], 1.0)
        proj = jax.lax.dot_general(
            feats, w_ref[...],
            dimension_numbers=(((1,), (1,)), ((), ())),
            preferred_element_type=jnp.float32,
        )
        out_ref[0] = proj + b_ref[...]
        lane = jax.lax.broadcasted_iota(jnp.int32, (1, MAX_PATCHES), 1)
        mask_ref[0] = ((num - MAX_PATCHES + lane) >= 0).astype(jnp.int32)


@jax.jit
def kernel(x, time_delta, W, b):
    B = x.shape[0]
    t = jnp.cumsum(time_delta, axis=1)
    ids = jnp.floor(t / PATCH_SIZE).astype(jnp.int32)

    ids_row = ids.reshape(B, 1, T)
    ids_chunk = ids.reshape(B, NCHUNK, 1, TCHUNK)
    b2 = b.reshape(1, D_MODEL)

    out, mask_i32 = pl.pallas_call(
        _patch_kernel,
        grid=(B, NCHUNK),
        in_specs=[
            pl.BlockSpec((1, 1, T), lambda i, c: (i, 0, 0)),
            pl.BlockSpec((1, 1, 1, TCHUNK), lambda i, c: (i, c, 0, 0)),
            pl.BlockSpec((1, TCHUNK, INPUT_DIM), lambda i, c: (i, c, 0)),
            pl.BlockSpec((D_MODEL, INPUT_DIM), lambda i, c: (0, 0)),
            pl.BlockSpec((1, D_MODEL), lambda i, c: (0, 0)),
        ],
        out_specs=[
            pl.BlockSpec((1, MAX_PATCHES, D_MODEL), lambda i, c: (i, 0, 0)),
            pl.BlockSpec((1, 1, MAX_PATCHES), lambda i, c: (i, 0, 0)),
        ],
        out_shape=[
            jax.ShapeDtypeStruct((B, MAX_PATCHES, D_MODEL), jnp.float32),
            jax.ShapeDtypeStruct((B, 1, MAX_PATCHES), jnp.int32),
        ],
        scratch_shapes=[
            pltpu.VMEM((MAX_PATCHES, INPUT_DIM), jnp.float32),
            pltpu.VMEM((MAX_PATCHES, 1), jnp.float32),
        ],
    )(ids_row, ids_chunk, x, W, b2)

    masks = mask_i32.reshape(B, MAX_PATCHES) != 0
    return out, masks
